# Initial kernel scaffold; baseline (speedup 1.0000x reference)
import jax
import jax.numpy as jnp
from jax.experimental import pallas as pl


def kernel(x, W1, W2, W3, W4, g1, b1, m1, v1, g2, b2, m2, v2, g3, b3, m3, v3, tn_w, tn_b, tn_m, tn_v):
    raise NotImplementedError("write your pallas kernel here")


# speedup vs baseline: 1.2010x; 1.2010x over previous
import jax
import jax.numpy as jnp
from jax.experimental import pallas as pl
from jax.experimental.pallas import tpu as pltpu

BN_EPS = 1e-5
TN_EPS = 1e-4

_BM = 2048


def _sign_pm1(x, dtype):
    return jnp.where(x >= 0, 1.0, -1.0).astype(dtype)


def _binarize_weights_body(w1_ref, w2_ref, w3_ref, w4_ref, o1_ref, o2_ref, o3_ref, o4_ref):
    o1_ref[...] = _sign_pm1(w1_ref[...], jnp.bfloat16)
    o2_ref[...] = _sign_pm1(w2_ref[...], jnp.bfloat16)
    o3_ref[...] = _sign_pm1(w3_ref[...], jnp.bfloat16)
    o4_ref[...] = _sign_pm1(w4_ref[...], jnp.bfloat16)


def _mlp_body(tn_ref, x_ref, w1_ref, w2_ref, w3_ref, w4_ref, bn_ref, o_ref):
    h = _sign_pm1(2.0 * x_ref[...] - 1.0, jnp.bfloat16)
    for i, w_ref in enumerate((w1_ref, w2_ref, w3_ref)):
        y = jax.lax.dot_general(
            h, w_ref[...], (((1,), (1,)), ((), ())),
            preferred_element_type=jnp.float32)
        g = bn_ref[4 * i + 0, :]
        b = bn_ref[4 * i + 1, :]
        m = bn_ref[4 * i + 2, :]
        v = bn_ref[4 * i + 3, :]
        t = (y - m) * (g * jax.lax.rsqrt(v + BN_EPS)) + b
        h = _sign_pm1(t, jnp.bfloat16)
    y = jax.lax.dot_general(
        h, w4_ref[...], (((1,), (1,)), ((), ())),
        preferred_element_type=jnp.float32)
    tn_w, tn_b, tn_m, tn_v = tn_ref[0], tn_ref[1], tn_ref[2], tn_ref[3]
    o_ref[...] = (y - tn_m) * jax.lax.rsqrt(tn_v + TN_EPS) * tn_w + tn_b


def kernel(x, W1, W2, W3, W4, g1, b1, m1, v1, g2, b2, m2, v2, g3, b3, m3, v3,
           tn_w, tn_b, tn_m, tn_v):
    B, D = x.shape
    H = W1.shape[0]
    C = W4.shape[0]

    wb1, wb2, wb3, wb4 = pl.pallas_call(
        _binarize_weights_body,
        out_shape=[
            jax.ShapeDtypeStruct(W1.shape, jnp.bfloat16),
            jax.ShapeDtypeStruct(W2.shape, jnp.bfloat16),
            jax.ShapeDtypeStruct(W3.shape, jnp.bfloat16),
            jax.ShapeDtypeStruct(W4.shape, jnp.bfloat16),
        ],
        name="binarize_weights",
    )(W1, W2, W3, W4)

    bn = jnp.stack([g1, b1, m1, v1, g2, b2, m2, v2, g3, b3, m3, v3])
    tn = jnp.stack([tn_w, tn_b, tn_m, tn_v])

    bm = _BM if B % _BM == 0 else B
    grid = (B // bm,)
    out = pl.pallas_call(
        _mlp_body,
        grid=grid,
        in_specs=[
            pl.BlockSpec(memory_space=pltpu.SMEM),
            pl.BlockSpec((bm, D), lambda i: (i, 0)),
            pl.BlockSpec((H, D), lambda i: (0, 0)),
            pl.BlockSpec((H, H), lambda i: (0, 0)),
            pl.BlockSpec((H, H), lambda i: (0, 0)),
            pl.BlockSpec((C, H), lambda i: (0, 0)),
            pl.BlockSpec((12, H), lambda i: (0, 0)),
        ],
        out_specs=pl.BlockSpec((bm, C), lambda i: (i, 0)),
        out_shape=jax.ShapeDtypeStruct((B, C), jnp.float32),
        compiler_params=pltpu.CompilerParams(
            dimension_semantics=("parallel",),
        ),
        name="binary_mlp_fused",
    )(tn, x, wb1, wb2, wb3, wb4, bn)
    return out


# speedup vs baseline: 1.6053x; 1.3366x over previous
import jax
import jax.numpy as jnp
from jax.experimental import pallas as pl
from jax.experimental.pallas import tpu as pltpu

BN_EPS = 1e-5
TN_EPS = 1e-4

_BM = 2048


def _sign_pm1(x, dtype):
    return jnp.where(x >= 0, 1.0, -1.0).astype(dtype)


_MM_DTYPE = jnp.float8_e4m3fn


def _binarize_weights_body(w1_ref, w2_ref, w3_ref, w4_ref, o1_ref, o2_ref, o3_ref, o4_ref):
    o1_ref[...] = _sign_pm1(w1_ref[...], _MM_DTYPE)
    o2_ref[...] = _sign_pm1(w2_ref[...], _MM_DTYPE)
    o3_ref[...] = _sign_pm1(w3_ref[...], _MM_DTYPE)
    o4_ref[...] = _sign_pm1(w4_ref[...], _MM_DTYPE)


def _mlp_body(tn_ref, x_ref, w1_ref, w2_ref, w3_ref, w4_ref, bn_ref, o_ref):
    h = _sign_pm1(2.0 * x_ref[...] - 1.0, _MM_DTYPE)
    for i, w_ref in enumerate((w1_ref, w2_ref, w3_ref)):
        y = jax.lax.dot_general(
            h, w_ref[...], (((1,), (1,)), ((), ())),
            preferred_element_type=jnp.float32)
        g = bn_ref[4 * i + 0, :]
        b = bn_ref[4 * i + 1, :]
        m = bn_ref[4 * i + 2, :]
        v = bn_ref[4 * i + 3, :]
        t = (y - m) * (g * jax.lax.rsqrt(v + BN_EPS)) + b
        h = _sign_pm1(t, _MM_DTYPE)
    y = jax.lax.dot_general(
        h, w4_ref[...], (((1,), (1,)), ((), ())),
        preferred_element_type=jnp.float32)
    tn_w, tn_b, tn_m, tn_v = tn_ref[0], tn_ref[1], tn_ref[2], tn_ref[3]
    o_ref[...] = (y - tn_m) * jax.lax.rsqrt(tn_v + TN_EPS) * tn_w + tn_b


def kernel(x, W1, W2, W3, W4, g1, b1, m1, v1, g2, b2, m2, v2, g3, b3, m3, v3,
           tn_w, tn_b, tn_m, tn_v):
    B, D = x.shape
    H = W1.shape[0]
    C = W4.shape[0]

    wb1, wb2, wb3, wb4 = pl.pallas_call(
        _binarize_weights_body,
        out_shape=[
            jax.ShapeDtypeStruct(W1.shape, _MM_DTYPE),
            jax.ShapeDtypeStruct(W2.shape, _MM_DTYPE),
            jax.ShapeDtypeStruct(W3.shape, _MM_DTYPE),
            jax.ShapeDtypeStruct(W4.shape, _MM_DTYPE),
        ],
        name="binarize_weights",
    )(W1, W2, W3, W4)

    bn = jnp.stack([g1, b1, m1, v1, g2, b2, m2, v2, g3, b3, m3, v3])
    tn = jnp.stack([tn_w, tn_b, tn_m, tn_v])

    bm = _BM if B % _BM == 0 else B
    grid = (B // bm,)
    out = pl.pallas_call(
        _mlp_body,
        grid=grid,
        in_specs=[
            pl.BlockSpec(memory_space=pltpu.SMEM),
            pl.BlockSpec((bm, D), lambda i: (i, 0)),
            pl.BlockSpec((H, D), lambda i: (0, 0)),
            pl.BlockSpec((H, H), lambda i: (0, 0)),
            pl.BlockSpec((H, H), lambda i: (0, 0)),
            pl.BlockSpec((C, H), lambda i: (0, 0)),
            pl.BlockSpec((12, H), lambda i: (0, 0)),
        ],
        out_specs=pl.BlockSpec((bm, C), lambda i: (i, 0)),
        out_shape=jax.ShapeDtypeStruct((B, C), jnp.float32),
        compiler_params=pltpu.CompilerParams(
            dimension_semantics=("parallel",),
        ),
        name="binary_mlp_fused",
    )(tn, x, wb1, wb2, wb3, wb4, bn)
    return out


# speedup vs baseline: 1.6559x; 1.0315x over previous
import jax
import jax.numpy as jnp
from jax.experimental import pallas as pl
from jax.experimental.pallas import tpu as pltpu

BN_EPS = 1e-5
TN_EPS = 1e-4

_BM = 2048


def _sign_pm1(x, dtype):
    return jnp.where(x >= 0, 1.0, -1.0).astype(dtype)


_MM_DTYPE = jnp.float8_e4m3fn


def _binarize_weights_body(w1_ref, w2_ref, w3_ref, w4_ref, o1_ref, o2_ref, o3_ref, o4_ref):
    o1_ref[...] = _sign_pm1(w1_ref[...], _MM_DTYPE)
    o2_ref[...] = _sign_pm1(w2_ref[...], _MM_DTYPE)
    o3_ref[...] = _sign_pm1(w3_ref[...], _MM_DTYPE)
    o4_ref[...] = _sign_pm1(w4_ref[...], _MM_DTYPE)


def _mlp_body(tn_ref, x_ref, w1_ref, w2_ref, w3_ref, w4_ref, bn_ref, o_ref):
    h = jnp.where(x_ref[...] >= 0.5, 1.0, -1.0).astype(_MM_DTYPE)
    for i, w_ref in enumerate((w1_ref, w2_ref, w3_ref)):
        y = jax.lax.dot_general(
            h, w_ref[...], (((1,), (1,)), ((), ())),
            preferred_element_type=jnp.float32)
        g = bn_ref[4 * i + 0, :]
        b = bn_ref[4 * i + 1, :]
        m = bn_ref[4 * i + 2, :]
        v = bn_ref[4 * i + 3, :]
        t = (y - m) * (g * jax.lax.rsqrt(v + BN_EPS)) + b
        h = _sign_pm1(t, _MM_DTYPE)
    y = jax.lax.dot_general(
        h, w4_ref[...], (((1,), (1,)), ((), ())),
        preferred_element_type=jnp.float32)
    tn_w, tn_b, tn_m, tn_v = tn_ref[0], tn_ref[1], tn_ref[2], tn_ref[3]
    o_ref[...] = (y - tn_m) * jax.lax.rsqrt(tn_v + TN_EPS) * tn_w + tn_b


def kernel(x, W1, W2, W3, W4, g1, b1, m1, v1, g2, b2, m2, v2, g3, b3, m3, v3,
           tn_w, tn_b, tn_m, tn_v):
    B, D = x.shape
    H = W1.shape[0]
    C = W4.shape[0]

    wb1, wb2, wb3, wb4 = pl.pallas_call(
        _binarize_weights_body,
        out_shape=[
            jax.ShapeDtypeStruct(W1.shape, _MM_DTYPE),
            jax.ShapeDtypeStruct(W2.shape, _MM_DTYPE),
            jax.ShapeDtypeStruct(W3.shape, _MM_DTYPE),
            jax.ShapeDtypeStruct(W4.shape, _MM_DTYPE),
        ],
        name="binarize_weights",
    )(W1, W2, W3, W4)

    bn = jnp.stack([g1, b1, m1, v1, g2, b2, m2, v2, g3, b3, m3, v3])
    tn = jnp.stack([tn_w, tn_b, tn_m, tn_v])

    bm = _BM if B % _BM == 0 else B
    grid = (B // bm,)
    out = pl.pallas_call(
        _mlp_body,
        grid=grid,
        in_specs=[
            pl.BlockSpec(memory_space=pltpu.SMEM),
            pl.BlockSpec((bm, D), lambda i: (i, 0)),
            pl.BlockSpec((H, D), lambda i: (0, 0)),
            pl.BlockSpec((H, H), lambda i: (0, 0)),
            pl.BlockSpec((H, H), lambda i: (0, 0)),
            pl.BlockSpec((C, H), lambda i: (0, 0)),
            pl.BlockSpec((12, H), lambda i: (0, 0)),
        ],
        out_specs=pl.BlockSpec((bm, C), lambda i: (i, 0)),
        out_shape=jax.ShapeDtypeStruct((B, C), jnp.float32),
        compiler_params=pltpu.CompilerParams(
            dimension_semantics=("parallel",),
        ),
        name="binary_mlp_fused",
    )(tn, x, wb1, wb2, wb3, wb4, bn)
    return out


# speedup vs baseline: 1.8732x; 1.1312x over previous
import jax
import jax.numpy as jnp
from jax.experimental import pallas as pl
from jax.experimental.pallas import tpu as pltpu

BN_EPS = 1e-5
TN_EPS = 1e-4

_MM_DTYPE = jnp.float8_e4m3fn

_BM = 1024
_SLAB = 512
_POP_ROWS = 64
_T = 256


def _sign_pm1(x, dtype):
    return jnp.where(x >= 0, 1.0, -1.0).astype(dtype)


def _binarize_weights_body(w1_ref, w2_ref, w3_ref, w4_ref,
                           o1_ref, o2_ref, o3_ref, o4_ref):
    o1_ref[...] = _sign_pm1(w1_ref[...], _MM_DTYPE)
    o2_ref[...] = _sign_pm1(w2_ref[...], _MM_DTYPE)
    o3_ref[...] = _sign_pm1(w3_ref[...], _MM_DTYPE)
    o4_ref[...] = _sign_pm1(w4_ref[...], _MM_DTYPE)


def _layer_matmul(h_ref, w_ref, out_fn, n_blocks_per_mxu):
    k_tiles = h_ref.shape[1] // _T
    slabs = _BM // _SLAB
    chunks = _SLAB // _POP_ROWS
    for n_local in range(n_blocks_per_mxu):
        for mxu in range(2):
            n = 2 * n_local + mxu if n_blocks_per_mxu > 1 else mxu
            for k in range(k_tiles):
                pltpu.matmul_push_rhs(
                    w_ref[n * _T:(n + 1) * _T, k * _T:(k + 1) * _T],
                    staging_register=k % 2, mxu_index=mxu, transpose=True)
                for s in range(slabs):
                    pltpu.matmul_acc_lhs(
                        acc_addr=s * (_SLAB // 4),
                        lhs=h_ref[s * _SLAB:(s + 1) * _SLAB,
                                  k * _T:(k + 1) * _T],
                        mxu_index=mxu,
                        load_staged_rhs=(k % 2) if s == 0 else None)
            for s in range(slabs):
                for c in range(chunks):
                    y = pltpu.matmul_pop(
                        acc_addr=s * (_SLAB // 4) + c * (_POP_ROWS // 4),
                        shape=(_POP_ROWS, _T), dtype=jnp.float32,
                        mxu_index=mxu)
                    rows = s * _SLAB + c * _POP_ROWS
                    out_fn(rows, n * _T, y)


def _layer4_matmul(h_ref, w_ref, out_fn):
    k_tiles = h_ref.shape[1] // _T
    chunks = _SLAB // _POP_ROWS
    for k in range(k_tiles):
        for mxu in range(2):
            pltpu.matmul_push_rhs(
                w_ref[:, k * _T:(k + 1) * _T],
                staging_register=k % 2, mxu_index=mxu, transpose=True)
            pltpu.matmul_acc_lhs(
                acc_addr=0,
                lhs=h_ref[mxu * _SLAB:(mxu + 1) * _SLAB,
                          k * _T:(k + 1) * _T],
                mxu_index=mxu,
                load_staged_rhs=k % 2)
    for mxu in range(2):
        for c in range(chunks):
            y = pltpu.matmul_pop(
                acc_addr=c * (_POP_ROWS // 4),
                shape=(_POP_ROWS, _T), dtype=jnp.float32, mxu_index=mxu)
            out_fn(mxu * _SLAB + c * _POP_ROWS, 0, y)


def _mlp_body(tn_ref, x_ref, w1_ref, w2_ref, w3_ref, w4_ref, bn_ref, o_ref,
              ha_ref, hb_ref):
    ha_ref[...] = jnp.where(x_ref[...] >= 0.5, 1.0, -1.0).astype(_MM_DTYPE)

    bufs = (ha_ref, hb_ref)
    for i, w_ref in enumerate((w1_ref, w2_ref, w3_ref)):
        h_in = bufs[i % 2]
        h_out = bufs[(i + 1) % 2]

        def bn_sign(rows, cols, y, i=i, h_out=h_out):
            g = bn_ref[4 * i + 0, cols:cols + _T]
            b = bn_ref[4 * i + 1, cols:cols + _T]
            m = bn_ref[4 * i + 2, cols:cols + _T]
            v = bn_ref[4 * i + 3, cols:cols + _T]
            t = (y - m) * (g * jax.lax.rsqrt(v + BN_EPS)) + b
            h_out[rows:rows + _POP_ROWS, cols:cols + _T] = _sign_pm1(t, _MM_DTYPE)

        _layer_matmul(h_in, w_ref, bn_sign, n_blocks_per_mxu=2)

    h_last = bufs[1]
    tn_w, tn_b, tn_m, tn_v = tn_ref[0], tn_ref[1], tn_ref[2], tn_ref[3]
    c_out = o_ref.shape[1]

    def tensor_norm(rows, cols, y, ):
        del cols
        yc = y[:, :c_out]
        o_ref[rows:rows + _POP_ROWS, :] = (
            (yc - tn_m) * jax.lax.rsqrt(tn_v + TN_EPS) * tn_w + tn_b)

    _layer4_matmul(h_last, w4_ref, tensor_norm)


def kernel(x, W1, W2, W3, W4, g1, b1, m1, v1, g2, b2, m2, v2, g3, b3, m3, v3,
           tn_w, tn_b, tn_m, tn_v):
    B, D = x.shape
    H = W1.shape[0]
    C = W4.shape[0]

    w4_padded = jnp.zeros((_T, H), jnp.float32).at[:C].set(W4)
    wb1, wb2, wb3, wb4 = pl.pallas_call(
        _binarize_weights_body,
        out_shape=[
            jax.ShapeDtypeStruct(W1.shape, _MM_DTYPE),
            jax.ShapeDtypeStruct(W2.shape, _MM_DTYPE),
            jax.ShapeDtypeStruct(W3.shape, _MM_DTYPE),
            jax.ShapeDtypeStruct((_T, H), _MM_DTYPE),
        ],
        name="binarize_weights",
    )(W1, W2, W3, w4_padded)

    bn = jnp.stack([g1, b1, m1, v1, g2, b2, m2, v2, g3, b3, m3, v3])
    tn = jnp.stack([tn_w, tn_b, tn_m, tn_v])

    grid = (B // _BM,)
    out = pl.pallas_call(
        _mlp_body,
        grid=grid,
        in_specs=[
            pl.BlockSpec(memory_space=pltpu.SMEM),
            pl.BlockSpec((_BM, D), lambda i: (i, 0)),
            pl.BlockSpec((H, D), lambda i: (0, 0)),
            pl.BlockSpec((H, H), lambda i: (0, 0)),
            pl.BlockSpec((H, H), lambda i: (0, 0)),
            pl.BlockSpec((_T, H), lambda i: (0, 0)),
            pl.BlockSpec((12, H), lambda i: (0, 0)),
        ],
        out_specs=pl.BlockSpec((_BM, C), lambda i: (i, 0)),
        out_shape=jax.ShapeDtypeStruct((B, C), jnp.float32),
        scratch_shapes=[
            pltpu.VMEM((_BM, D), _MM_DTYPE),
            pltpu.VMEM((_BM, H), _MM_DTYPE),
        ],
        compiler_params=pltpu.CompilerParams(
            dimension_semantics=("parallel",),
        ),
        name="binary_mlp_fused",
    )(tn, x, wb1, wb2, wb3, wb4, bn)
    return out
